# initial kernel scaffold (unmeasured)
import jax
import jax.numpy as jnp
from jax import lax
from jax.experimental import pallas as pl
from jax.experimental.pallas import tpu as pltpu

N_DEV = 4
N_TOK = 2048
D = 1024
H = 1024
E_LOCAL = 8
CHUNK = N_TOK // N_DEV


def kernel(x, router_W, route_idx, expert_W, shared_W):
    def body(x_ref, router_W_ref, route_idx_ref, expert_W_ref, shared_W_ref,
             out_ref, wbuf, comm_ref, dma_sems, send_sems, recv_sems):
        my = lax.axis_index("i")
        left = lax.rem(my - 1 + N_DEV, N_DEV)
        right = lax.rem(my + 1, N_DEV)

        barrier_sem = pltpu.get_barrier_semaphore()
        for nbr in (left, right):
            pl.semaphore_signal(
                barrier_sem, inc=1,
                device_id=(nbr,), device_id_type=pl.DeviceIdType.MESH,
            )
        pl.semaphore_wait(barrier_sem, 2)

        xf = x_ref[:, :]
        scores = jnp.dot(xf, router_W_ref[:, :],
                         preferred_element_type=jnp.float32)
        s_max = jnp.max(scores, axis=-1, keepdims=True)
        p_top = 1.0 / jnp.sum(jnp.exp(scores - s_max), axis=-1,
                              keepdims=True)
        ridx = route_idx_ref[:, :]

        xb = xf.astype(jnp.bfloat16)
        first = my * E_LOCAL

        cp0 = pltpu.make_async_copy(expert_W_ref.at[0], wbuf.at[0],
                                    dma_sems.at[0])
        cp0.start()
        for j in range(E_LOCAL):
            if j + 1 < E_LOCAL:
                nxt = pltpu.make_async_copy(expert_W_ref.at[j + 1],
                                            wbuf.at[(j + 1) % 2],
                                            dma_sems.at[(j + 1) % 2])
                nxt.start()
            pltpu.make_async_copy(expert_W_ref.at[j], wbuf.at[j % 2],
                                  dma_sems.at[j % 2]).wait()
            wj = jnp.where(ridx == first + j, p_top, 0.0)
            y = jnp.dot(xb, wbuf[j % 2].astype(jnp.bfloat16),
                        preferred_element_type=jnp.float32)
            if j == 0:
                out_ref[:, :] = wj * y
            else:
                out_ref[:, :] += wj * y

        rows = pl.ds(my * CHUNK, CHUNK)
        shared_b = shared_W_ref[:, :].astype(jnp.bfloat16)
        out_ref[rows, :] += jnp.dot(xb[my * CHUNK:(my + 1) * CHUNK
                                       if False else pl.ds(0, 0), :]
                                    if False else
                                    lax.dynamic_slice(xb, (my * CHUNK, 0),
                                                      (CHUNK, D)),
                                    shared_b,
                                    preferred_element_type=jnp.float32)

        for s in range(N_DEV - 1):
            c_send = lax.rem(my - s + N_DEV, N_DEV)
            c_recv = lax.rem(my - s - 1 + N_DEV, N_DEV)
            rdma = pltpu.make_async_remote_copy(
                src_ref=out_ref.at[pl.ds(c_send * CHUNK, CHUNK), :],
                dst_ref=comm_ref.at[s],
                send_sem=send_sems.at[s],
                recv_sem=recv_sems.at[s],
                device_id=(right,),
                device_id_type=pl.DeviceIdType.MESH,
            )
            rdma.start()
            rdma.wait()
            out_ref[pl.ds(c_recv * CHUNK, CHUNK), :] += comm_ref[s, :, :]

        for s in range(N_DEV - 1):
            c_send = lax.rem(my + 1 - s + N_DEV, N_DEV)
            c_recv = lax.rem(my - s + N_DEV, N_DEV)
            rdma = pltpu.make_async_remote_copy(
                src_ref=out_ref.at[pl.ds(c_send * CHUNK, CHUNK), :],
                dst_ref=out_ref.at[pl.ds(c_send * CHUNK, CHUNK), :],
                send_sem=send_sems.at[N_DEV - 1 + s],
                recv_sem=recv_sems.at[N_DEV - 1 + s],
                device_id=(right,),
                device_id_type=pl.DeviceIdType.MESH,
            )
            rdma.start()
            rdma.wait()

    out_shape = jax.ShapeDtypeStruct((N_TOK, H), jnp.float32)
    return pl.pallas_call(
        body,
        out_shape=out_shape,
        in_specs=[
            pl.BlockSpec(memory_space=pltpu.VMEM),
            pl.BlockSpec(memory_space=pltpu.VMEM),
            pl.BlockSpec(memory_space=pltpu.VMEM),
            pl.BlockSpec(memory_space=pltpu.ANY),
            pl.BlockSpec(memory_space=pltpu.VMEM),
        ],
        out_specs=pl.BlockSpec(memory_space=pltpu.VMEM),
        scratch_shapes=[
            pltpu.VMEM((2, D, H), jnp.float32),
            pltpu.VMEM((N_DEV - 1, CHUNK, H), jnp.float32),
            pltpu.SemaphoreType.DMA((2,)),
            pltpu.SemaphoreType.DMA((2 * (N_DEV - 1),)),
            pltpu.SemaphoreType.DMA((2 * (N_DEV - 1),)),
        ],
        compiler_params=pltpu.CompilerParams(collective_id=0),
    )(x, router_W, route_idx, expert_W, shared_W)


# baseline (device time: 210530 ns/iter reference)
import jax
import jax.numpy as jnp
from jax import lax
from jax.experimental import pallas as pl
from jax.experimental.pallas import tpu as pltpu

N_DEV = 4
N_TOK = 2048
D = 1024
H = 1024
E_LOCAL = 8
CHUNK = N_TOK // N_DEV


def kernel(x, router_W, route_idx, expert_W, shared_W):
    def body(x_ref, router_W_ref, route_idx_ref, expert_W_ref, shared_W_ref,
             out_ref, wbuf, comm_ref, dma_sems, send_sems, recv_sems):
        my = lax.axis_index("i")
        left = lax.rem(my - 1 + N_DEV, N_DEV)
        right = lax.rem(my + 1, N_DEV)

        barrier_sem = pltpu.get_barrier_semaphore()
        for nbr in (left, right):
            pl.semaphore_signal(
                barrier_sem, inc=1,
                device_id=(nbr,), device_id_type=pl.DeviceIdType.MESH,
            )
        pl.semaphore_wait(barrier_sem, 2)

        xf = x_ref[:, :]
        scores = jnp.dot(xf, router_W_ref[:, :],
                         preferred_element_type=jnp.float32)
        s_max = jnp.max(scores, axis=-1, keepdims=True)
        p_top = 1.0 / jnp.sum(jnp.exp(scores - s_max), axis=-1,
                              keepdims=True)
        ridx = route_idx_ref[:, :]

        xb = xf.astype(jnp.bfloat16)
        first = my * E_LOCAL

        cp0 = pltpu.make_async_copy(expert_W_ref.at[0], wbuf.at[0],
                                    dma_sems.at[0])
        cp0.start()
        for j in range(E_LOCAL):
            if j + 1 < E_LOCAL:
                nxt = pltpu.make_async_copy(expert_W_ref.at[j + 1],
                                            wbuf.at[(j + 1) % 2],
                                            dma_sems.at[(j + 1) % 2])
                nxt.start()
            pltpu.make_async_copy(expert_W_ref.at[j], wbuf.at[j % 2],
                                  dma_sems.at[j % 2]).wait()
            wj = jnp.where(ridx == first + j, p_top, 0.0)
            y = jnp.dot(xb, wbuf[j % 2].astype(jnp.bfloat16),
                        preferred_element_type=jnp.float32)
            if j == 0:
                out_ref[:, :] = wj * y
            else:
                out_ref[:, :] += wj * y

        rows = pl.ds(my * CHUNK, CHUNK)
        shared_b = shared_W_ref[:, :].astype(jnp.bfloat16)
        x_chunk = x_ref[rows, :].astype(jnp.bfloat16)
        out_ref[rows, :] += jnp.dot(x_chunk, shared_b,
                                    preferred_element_type=jnp.float32)

        for s in range(N_DEV - 1):
            c_send = lax.rem(my - s + N_DEV, N_DEV)
            c_recv = lax.rem(my - s - 1 + N_DEV, N_DEV)
            rdma = pltpu.make_async_remote_copy(
                src_ref=out_ref.at[pl.ds(c_send * CHUNK, CHUNK), :],
                dst_ref=comm_ref.at[s],
                send_sem=send_sems.at[s],
                recv_sem=recv_sems.at[s],
                device_id=(right,),
                device_id_type=pl.DeviceIdType.MESH,
            )
            rdma.start()
            rdma.wait()
            out_ref[pl.ds(c_recv * CHUNK, CHUNK), :] += comm_ref[s, :, :]

        for s in range(N_DEV - 1):
            c_send = lax.rem(my + 1 - s + N_DEV, N_DEV)
            c_recv = lax.rem(my - s + N_DEV, N_DEV)
            rdma = pltpu.make_async_remote_copy(
                src_ref=out_ref.at[pl.ds(c_send * CHUNK, CHUNK), :],
                dst_ref=out_ref.at[pl.ds(c_send * CHUNK, CHUNK), :],
                send_sem=send_sems.at[N_DEV - 1 + s],
                recv_sem=recv_sems.at[N_DEV - 1 + s],
                device_id=(right,),
                device_id_type=pl.DeviceIdType.MESH,
            )
            rdma.start()
            rdma.wait()

    out_shape = jax.ShapeDtypeStruct((N_TOK, H), jnp.float32)
    return pl.pallas_call(
        body,
        out_shape=out_shape,
        in_specs=[
            pl.BlockSpec(memory_space=pltpu.VMEM),
            pl.BlockSpec(memory_space=pltpu.VMEM),
            pl.BlockSpec(memory_space=pltpu.VMEM),
            pl.BlockSpec(memory_space=pl.ANY),
            pl.BlockSpec(memory_space=pltpu.VMEM),
        ],
        out_specs=pl.BlockSpec(memory_space=pltpu.VMEM),
        scratch_shapes=[
            pltpu.VMEM((2, D, H), jnp.float32),
            pltpu.VMEM((N_DEV - 1, CHUNK, H), jnp.float32),
            pltpu.SemaphoreType.DMA((2,)),
            pltpu.SemaphoreType.DMA((2 * (N_DEV - 1),)),
            pltpu.SemaphoreType.DMA((2 * (N_DEV - 1),)),
        ],
        compiler_params=pltpu.CompilerParams(collective_id=0),
    )(x, router_W, route_idx, expert_W, shared_W)


# device time: 111039 ns/iter; 1.8960x vs baseline; 1.8960x over previous
import jax
import jax.numpy as jnp
from jax import lax
from jax.experimental import pallas as pl
from jax.experimental.pallas import tpu as pltpu

N_DEV = 4
N_TOK = 2048
D = 1024
H = 1024
E_LOCAL = 8
CHUNK = N_TOK // N_DEV
HC = H // 2


def kernel(x, router_W, route_idx, expert_W, shared_W):
    def body(x_ref, router_W_ref, route_idx_ref, expert_W_ref, shared_W_ref,
             out_ref, wbuf, sbufR, sbufL, rsR, rsL, agR, agL,
             dma_sems, send_sems, recv_sems):
        my = lax.axis_index("i")
        left = lax.rem(my - 1 + N_DEV, N_DEV)
        right = lax.rem(my + 1, N_DEV)

        barrier_sem = pltpu.get_barrier_semaphore()
        for nbr in (left, right):
            pl.semaphore_signal(
                barrier_sem, inc=1,
                device_id=(nbr,), device_id_type=pl.DeviceIdType.MESH,
            )
        pl.semaphore_wait(barrier_sem, 2)

        xf = x_ref[:, :]
        scores = jnp.dot(xf, router_W_ref[:, :],
                         preferred_element_type=jnp.float32)
        s_max = jnp.max(scores, axis=-1, keepdims=True)
        p_top = 1.0 / jnp.sum(jnp.exp(scores - s_max), axis=-1,
                              keepdims=True)
        ridx = route_idx_ref[:, :]

        xb = xf.astype(jnp.bfloat16)
        first = my * E_LOCAL

        cp0 = pltpu.make_async_copy(expert_W_ref.at[0], wbuf.at[0],
                                    dma_sems.at[0])
        cp0.start()
        for j in range(E_LOCAL):
            if j + 1 < E_LOCAL:
                nxt = pltpu.make_async_copy(expert_W_ref.at[j + 1],
                                            wbuf.at[(j + 1) % 2],
                                            dma_sems.at[(j + 1) % 2])
                nxt.start()
            pltpu.make_async_copy(expert_W_ref.at[j], wbuf.at[j % 2],
                                  dma_sems.at[j % 2]).wait()
            wj = jnp.where(ridx == first + j, p_top, 0.0)
            y = jnp.dot(xb, wbuf[j % 2].astype(jnp.bfloat16),
                        preferred_element_type=jnp.float32)
            if j == 0:
                out_ref[:, :] = wj * y
            else:
                out_ref[:, :] += wj * y

        rows = pl.ds(my * CHUNK, CHUNK)
        shared_b = shared_W_ref[:, :].astype(jnp.bfloat16)
        x_chunk = x_ref[rows, :].astype(jnp.bfloat16)
        out_ref[rows, :] += jnp.dot(x_chunk, shared_b,
                                    preferred_element_type=jnp.float32)

        def ring_rdma(src, dst, sem_idx, dst_dev):
            return pltpu.make_async_remote_copy(
                src_ref=src, dst_ref=dst,
                send_sem=send_sems.at[sem_idx],
                recv_sem=recv_sems.at[sem_idx],
                device_id=(dst_dev,),
                device_id_type=pl.DeviceIdType.MESH,
            )

        colR = pl.ds(0, HC)
        colL = pl.ds(HC, HC)

        for s in range(N_DEV - 1):
            c_sR = lax.rem(my - s + N_DEV, N_DEV)
            c_rR = lax.rem(my - s - 1 + N_DEV, N_DEV)
            c_sL = lax.rem(my + s, N_DEV)
            c_rL = lax.rem(my + s + 1, N_DEV)
            sbufR[s] = out_ref[pl.ds(c_sR * CHUNK, CHUNK), colR].astype(
                jnp.bfloat16)
            sbufL[s] = out_ref[pl.ds(c_sL * CHUNK, CHUNK), colL].astype(
                jnp.bfloat16)
            rR = ring_rdma(sbufR.at[s], rsR.at[s], s, right)
            rL = ring_rdma(sbufL.at[s], rsL.at[s], 3 + s, left)
            rR.start()
            rL.start()
            rR.wait()
            rL.wait()
            out_ref[pl.ds(c_rR * CHUNK, CHUNK), colR] += rsR[s].astype(
                jnp.float32)
            out_ref[pl.ds(c_rL * CHUNK, CHUNK), colL] += rsL[s].astype(
                jnp.float32)

        own_R = lax.rem(my + 1, N_DEV)
        own_L = lax.rem(my - 1 + N_DEV, N_DEV)
        agR[3] = out_ref[pl.ds(own_R * CHUNK, CHUNK), colR].astype(
            jnp.bfloat16)
        agL[3] = out_ref[pl.ds(own_L * CHUNK, CHUNK), colL].astype(
            jnp.bfloat16)
        for s in range(N_DEV - 1):
            c_rR = lax.rem(my - s + N_DEV, N_DEV)
            c_rL = lax.rem(my + s, N_DEV)
            src_slot = 3 if s == 0 else s - 1
            rR = ring_rdma(agR.at[src_slot], agR.at[s], 6 + s, right)
            rL = ring_rdma(agL.at[src_slot], agL.at[s], 9 + s, left)
            rR.start()
            rL.start()
            rR.wait()
            rL.wait()
            out_ref[pl.ds(c_rR * CHUNK, CHUNK), colR] = agR[s].astype(
                jnp.float32)
            out_ref[pl.ds(c_rL * CHUNK, CHUNK), colL] = agL[s].astype(
                jnp.float32)

    out_shape = jax.ShapeDtypeStruct((N_TOK, H), jnp.float32)
    return pl.pallas_call(
        body,
        out_shape=out_shape,
        in_specs=[
            pl.BlockSpec(memory_space=pltpu.VMEM),
            pl.BlockSpec(memory_space=pltpu.VMEM),
            pl.BlockSpec(memory_space=pltpu.VMEM),
            pl.BlockSpec(memory_space=pl.ANY),
            pl.BlockSpec(memory_space=pltpu.VMEM),
        ],
        out_specs=pl.BlockSpec(memory_space=pltpu.VMEM),
        scratch_shapes=[
            pltpu.VMEM((2, D, H), jnp.float32),
            pltpu.VMEM((3, CHUNK, HC), jnp.bfloat16),
            pltpu.VMEM((3, CHUNK, HC), jnp.bfloat16),
            pltpu.VMEM((3, CHUNK, HC), jnp.bfloat16),
            pltpu.VMEM((3, CHUNK, HC), jnp.bfloat16),
            pltpu.VMEM((4, CHUNK, HC), jnp.bfloat16),
            pltpu.VMEM((4, CHUNK, HC), jnp.bfloat16),
            pltpu.SemaphoreType.DMA((2,)),
            pltpu.SemaphoreType.DMA((12,)),
            pltpu.SemaphoreType.DMA((12,)),
        ],
        compiler_params=pltpu.CompilerParams(collective_id=0),
    )(x, router_W, route_idx, expert_W, shared_W)


# device time: 94393 ns/iter; 2.2304x vs baseline; 1.1763x over previous
import jax
import jax.numpy as jnp
from jax import lax
from jax.experimental import pallas as pl
from jax.experimental.pallas import tpu as pltpu

N_DEV = 4
N_TOK = 2048
D = 1024
H = 1024
E_LOCAL = 8
CHUNK = N_TOK // N_DEV
HC = H // 2


def kernel(x, router_W, route_idx, expert_W, shared_W):
    def body(x_ref, router_W_ref, route_idx_ref, expert_W_ref, shared_W_ref,
             out_ref, wbuf, Wb_ref, ptop_ref, sbufR, sbufL, rsR, rsL,
             agR, agL, dma_sems, send_sems, recv_sems):
        my = lax.axis_index("i")
        left = lax.rem(my - 1 + N_DEV, N_DEV)
        right = lax.rem(my + 1, N_DEV)
        first = my * E_LOCAL

        barrier_sem = pltpu.get_barrier_semaphore()
        for nbr in (left, right):
            pl.semaphore_signal(
                barrier_sem, inc=1,
                device_id=(nbr,), device_id_type=pl.DeviceIdType.MESH,
            )
        pl.semaphore_wait(barrier_sem, 2)

        cp0 = pltpu.make_async_copy(expert_W_ref.at[0], wbuf.at[0],
                                    dma_sems.at[0])
        cp0.start()

        scores = jnp.dot(x_ref[:, :], router_W_ref[:, :],
                         preferred_element_type=jnp.float32)
        s_max = jnp.max(scores, axis=-1, keepdims=True)
        ptop_ref[:, :] = 1.0 / jnp.sum(jnp.exp(scores - s_max), axis=-1,
                                       keepdims=True)

        rows_my = pl.ds(my * CHUNK, CHUNK)
        xm = x_ref[rows_my, :].astype(jnp.bfloat16)
        ridx_m = route_idx_ref[rows_my, :]
        p_m = ptop_ref[rows_my, :]
        acc = jnp.dot(xm, shared_W_ref[:, :].astype(jnp.bfloat16),
                      preferred_element_type=jnp.float32)
        for j in range(E_LOCAL):
            if j + 1 < E_LOCAL:
                pltpu.make_async_copy(expert_W_ref.at[j + 1],
                                      wbuf.at[(j + 1) % 2],
                                      dma_sems.at[(j + 1) % 2]).start()
            pltpu.make_async_copy(expert_W_ref.at[j], wbuf.at[j % 2],
                                  dma_sems.at[j % 2]).wait()
            Wbj = wbuf[j % 2].astype(jnp.bfloat16)
            Wb_ref[j] = Wbj
            wj = jnp.where(ridx_m == first + j, p_m, 0.0)
            acc += wj * jnp.dot(xm, Wbj, preferred_element_type=jnp.float32)
        out_ref[rows_my, :] = acc

        def compute_half(c, lo):
            rows = pl.ds(c * CHUNK, CHUNK)
            xc = x_ref[rows, :].astype(jnp.bfloat16)
            ridx_c = route_idx_ref[rows, :]
            p_c = ptop_ref[rows, :]
            acc = jnp.zeros((CHUNK, HC), jnp.float32)
            for j in range(E_LOCAL):
                wj = jnp.where(ridx_c == first + j, p_c, 0.0)
                acc += wj * jnp.dot(xc, Wb_ref[j, :, lo:lo + HC],
                                    preferred_element_type=jnp.float32)
            out_ref[rows, lo:lo + HC] = acc

        def ring_rdma(src, dst, sem_idx, dst_dev):
            return pltpu.make_async_remote_copy(
                src_ref=src, dst_ref=dst,
                send_sem=send_sems.at[sem_idx],
                recv_sem=recv_sems.at[sem_idx],
                device_id=(dst_dev,),
                device_id_type=pl.DeviceIdType.MESH,
            )

        colR = pl.ds(0, HC)
        colL = pl.ds(HC, HC)
        c_m1 = lax.rem(my - 1 + N_DEV, N_DEV)
        c_p1 = lax.rem(my + 1, N_DEV)
        c_p2 = lax.rem(my + 2, N_DEV)

        def rs_start(s, c_sR, c_sL):
            sbufR[s] = out_ref[pl.ds(c_sR * CHUNK, CHUNK), colR].astype(
                jnp.bfloat16)
            sbufL[s] = out_ref[pl.ds(c_sL * CHUNK, CHUNK), colL].astype(
                jnp.bfloat16)
            rR = ring_rdma(sbufR.at[s], rsR.at[s], s, right)
            rL = ring_rdma(sbufL.at[s], rsL.at[s], 3 + s, left)
            rR.start()
            rL.start()
            return rR, rL

        def rs_finish(rR, rL, s, c_rR, c_rL):
            rR.wait()
            rL.wait()
            out_ref[pl.ds(c_rR * CHUNK, CHUNK), colR] += rsR[s].astype(
                jnp.float32)
            out_ref[pl.ds(c_rL * CHUNK, CHUNK), colL] += rsL[s].astype(
                jnp.float32)

        r0 = rs_start(0, my, my)
        compute_half(c_m1, 0)
        compute_half(c_p1, HC)
        rs_finish(*r0, 0, c_m1, c_p1)

        r1 = rs_start(1, c_m1, c_p1)
        compute_half(c_p2, 0)
        compute_half(c_p2, HC)
        rs_finish(*r1, 1, c_p2, c_p2)

        r2 = rs_start(2, c_p2, c_p2)
        compute_half(c_p1, 0)
        compute_half(c_m1, HC)
        rs_finish(*r2, 2, c_p1, c_m1)

        agR[3] = out_ref[pl.ds(c_p1 * CHUNK, CHUNK), colR].astype(
            jnp.bfloat16)
        agL[3] = out_ref[pl.ds(c_m1 * CHUNK, CHUNK), colL].astype(
            jnp.bfloat16)
        for s in range(N_DEV - 1):
            c_rR = lax.rem(my - s + N_DEV, N_DEV)
            c_rL = lax.rem(my + s, N_DEV)
            src_slot = 3 if s == 0 else s - 1
            rR = ring_rdma(agR.at[src_slot], agR.at[s], 6 + s, right)
            rL = ring_rdma(agL.at[src_slot], agL.at[s], 9 + s, left)
            rR.start()
            rL.start()
            rR.wait()
            rL.wait()
            out_ref[pl.ds(c_rR * CHUNK, CHUNK), colR] = agR[s].astype(
                jnp.float32)
            out_ref[pl.ds(c_rL * CHUNK, CHUNK), colL] = agL[s].astype(
                jnp.float32)

    out_shape = jax.ShapeDtypeStruct((N_TOK, H), jnp.float32)
    return pl.pallas_call(
        body,
        out_shape=out_shape,
        in_specs=[
            pl.BlockSpec(memory_space=pltpu.VMEM),
            pl.BlockSpec(memory_space=pltpu.VMEM),
            pl.BlockSpec(memory_space=pltpu.VMEM),
            pl.BlockSpec(memory_space=pl.ANY),
            pl.BlockSpec(memory_space=pltpu.VMEM),
        ],
        out_specs=pl.BlockSpec(memory_space=pltpu.VMEM),
        scratch_shapes=[
            pltpu.VMEM((2, D, H), jnp.float32),
            pltpu.VMEM((E_LOCAL, D, H), jnp.bfloat16),
            pltpu.VMEM((N_TOK, 1), jnp.float32),
            pltpu.VMEM((3, CHUNK, HC), jnp.bfloat16),
            pltpu.VMEM((3, CHUNK, HC), jnp.bfloat16),
            pltpu.VMEM((3, CHUNK, HC), jnp.bfloat16),
            pltpu.VMEM((3, CHUNK, HC), jnp.bfloat16),
            pltpu.VMEM((4, CHUNK, HC), jnp.bfloat16),
            pltpu.VMEM((4, CHUNK, HC), jnp.bfloat16),
            pltpu.SemaphoreType.DMA((2,)),
            pltpu.SemaphoreType.DMA((12,)),
            pltpu.SemaphoreType.DMA((12,)),
        ],
        compiler_params=pltpu.CompilerParams(
            collective_id=0,
            vmem_limit_bytes=100 * 1024 * 1024,
        ),
    )(x, router_W, route_idx, expert_W, shared_W)
